# baseline (speedup 1.0000x reference)
import functools

import jax
import jax.numpy as jnp
from jax import lax
from jax.experimental import pallas as pl
from jax.experimental.pallas import tpu as pltpu
from jax.experimental.pallas import tpu_sc as plsc

N_OUT = 32768
N_IN = 2048
BS = 512
L = 16
BSC = 32
NHALF = 2
RB = 2048
NB = N_OUT // RB
BPH = NB // NHALF
CH = 1024
S = CH // L
NG = CH // L
NBOUNDS = 64


def _sc_sparse_matmul(inp2, packed, bounds, bkg, rest10):
    mesh = plsc.VectorSubcoreMesh(core_axis_name="c", subcore_axis_name="s")

    @functools.partial(
        pl.kernel,
        out_type=jax.ShapeDtypeStruct((BS, N_OUT), jnp.float32),
        mesh=mesh,
        compiler_params=pltpu.CompilerParams(
            needs_layout_passes=False,
            use_tc_tiling_on_sc=False,
        ),
        scratch_types=[
            pltpu.VMEM((BSC // 2, N_IN), jnp.int32),
            pltpu.VMEM((BSC, RB), jnp.float32),
            pltpu.VMEM((3, CH), jnp.int32),
            pltpu.VMEM((3, CH), jnp.int32),
            pltpu.VMEM((NBOUNDS,), jnp.int32),
            pltpu.VMEM((RB,), jnp.float32),
            pltpu.VMEM((BSC, L), jnp.float32),
            pltpu.SemaphoreType.DMA,
            pltpu.SemaphoreType.DMA,
        ],
    )
    def body(inp_hbm, packed_hbm, bounds_hbm, bkg_hbm, rest_hbm,
             out_hbm, inp_v, acc_v, chA_v, chB_v, bounds_v, bkg_v,
             restm_v, semA, semB):
        wid = lax.axis_index("s") * 2 + lax.axis_index("c")
        half = wid % NHALF
        bs0 = (wid // NHALF) * BSC

        pltpu.sync_copy(inp_hbm.at[pl.ds(bs0 // 2, BSC // 2), :], inp_v)
        pltpu.sync_copy(rest_hbm.at[pl.ds(bs0, BSC), :], restm_v)
        pltpu.sync_copy(bounds_hbm, bounds_v)


        def pick(g):
            return bounds_v[pl.ds(g, L)][0]

        def block_body(b, _):
            g = half * BPH + b
            base = g * RB
            s = pick(g)
            e = pick(g + 1)

            pltpu.sync_copy(bkg_hbm.at[pl.ds(base, RB)], bkg_v)

            def init_body(r, _):
                bk = bkg_v[pl.ds(r * L, L)]
                for j in range(BSC):
                    acc_v[j, pl.ds(r * L, L)] = bk * restm_v[j, :]
                return 0

            lax.fori_loop(0, RB // L, init_body, 0)

            def process_chunk(buf, t, masked):

                @plsc.parallel_loop(0, NG, 1, unroll=2)
                def group_body(k):
                    off = k * L
                    rows16 = buf[0, pl.ds(off, L)]
                    cols16 = buf[1, pl.ds(off, L)]
                    w16 = plsc.bitcast(buf[2, pl.ds(off, L)], jnp.float32)
                    rloc = rows16 - base
                    if masked:
                        valid = (rows16 >= base) & (rows16 < base + RB)
                    else:
                        valid = None
                    for j2 in range(BSC // 2):
                        v = plsc.load_gather(inp_v.at[j2], [cols16])
                        ge = plsc.bitcast(lax.shift_left(v, 16),
                                          jnp.float32)
                        go = plsc.bitcast(v & jnp.int32(-65536),
                                          jnp.float32)
                        plsc.addupdate_scatter(acc_v.at[2 * j2], [rloc],
                                               ge * w16, mask=valid)
                        plsc.addupdate_scatter(acc_v.at[2 * j2 + 1], [rloc],
                                               go * w16, mask=valid)

            @pl.when(e > s)
            def _():
                t0 = s // CH
                t1 = (e - 1) // CH
                ntc = t1 - t0 + 1
                pltpu.async_copy(packed_hbm.at[t0], chA_v, semA)

                def chunk_loop(ci, _):
                    t = t0 + ci

                    @pl.when(ci % 2 == 0)
                    def _():
                        pltpu.make_async_copy(packed_hbm.at[t0], chA_v,
                                              semA).wait()

                        @pl.when(t + 1 <= t1)
                        def _():
                            pltpu.async_copy(packed_hbm.at[t + 1], chB_v,
                                             semB)

                        bnd = (t == t0) | (t == t1)

                        @pl.when(bnd)
                        def _():
                            process_chunk(chA_v, t, True)

                        @pl.when(jnp.logical_not(bnd))
                        def _():
                            process_chunk(chA_v, t, False)

                    @pl.when(ci % 2 == 1)
                    def _():
                        pltpu.make_async_copy(packed_hbm.at[t0], chB_v,
                                              semB).wait()

                        @pl.when(t + 1 <= t1)
                        def _():
                            pltpu.async_copy(packed_hbm.at[t + 1], chA_v,
                                             semA)

                        bnd = (t == t0) | (t == t1)

                        @pl.when(bnd)
                        def _():
                            process_chunk(chB_v, t, True)

                        @pl.when(jnp.logical_not(bnd))
                        def _():
                            process_chunk(chB_v, t, False)

                    return 0

                lax.fori_loop(0, ntc, chunk_loop, 0)

            pltpu.sync_copy(acc_v,
                            out_hbm.at[pl.ds(bs0, BSC), pl.ds(base, RB)])
            return 0

        lax.fori_loop(0, BPH, block_body, 0)

    return body(inp2, packed, bounds, bkg, rest10)


def kernel(inp, indices, weights, bkg_weights):
    b, s, f = inp.shape
    inp2 = inp.reshape(b * s, f).astype(jnp.float32)
    ev = lax.bitcast_convert_type(
        inp2[0::2].astype(jnp.bfloat16), jnp.uint16).astype(jnp.uint32)
    od = lax.bitcast_convert_type(
        inp2[1::2].astype(jnp.bfloat16), jnp.uint16).astype(jnp.uint32)
    inp_pk = lax.bitcast_convert_type(ev | (od << 16), jnp.int32)
    rows = indices[:, 0].astype(jnp.int32)
    cols = indices[:, 1].astype(jnp.int32)
    w32 = weights.astype(jnp.float32)

    nnz = rows.shape[0]
    pad = (-nnz) % CH
    if pad:
        rows_p = jnp.pad(rows, (0, pad), constant_values=N_OUT - 1)
        cols_p = jnp.pad(cols, (0, pad))
        w_p = jnp.pad(w32, (0, pad))
    else:
        rows_p, cols_p, w_p = rows, cols, w32
    lane = jnp.arange(CH)
    perm = (lane % L) * S + lane // L
    w_i = lax.bitcast_convert_type(w_p, jnp.int32)
    packed = jnp.stack([rows_p.reshape(-1, CH)[:, perm],
                        cols_p.reshape(-1, CH)[:, perm],
                        w_i.reshape(-1, CH)[:, perm]], axis=1)

    edges = jnp.arange(0, N_OUT + 1, RB, dtype=jnp.int32)
    bounds = jnp.searchsorted(rows, edges, side="left").astype(jnp.int32)
    bounds = jnp.pad(bounds, (0, NBOUNDS - bounds.shape[0]))

    kn = jax.random.key(42)
    rest = jnp.sum((jax.random.uniform(kn, (b, s, 10)) < 0.1)
                   .astype(jnp.float32), -1).reshape(b * s)
    rest10 = jnp.broadcast_to((rest / 10.0)[:, None], (b * s, L))

    out2 = _sc_sparse_matmul(inp_pk, packed, bounds,
                             bkg_weights.astype(jnp.float32), rest10)
    return out2.reshape(b, s, N_OUT)


# speedup vs baseline: 1.0719x; 1.0719x over previous
import functools

import jax
import jax.numpy as jnp
from jax import lax
from jax.experimental import pallas as pl
from jax.experimental.pallas import tpu as pltpu
from jax.experimental.pallas import tpu_sc as plsc

N_OUT = 32768
N_IN = 2048
BS = 512
L = 16
BSC = 32
NHALF = 2
RB = 1024
NB = N_OUT // RB
BPH = NB // NHALF
CH = 512
S = CH // L
NG = CH // L
NBOUNDS = 64


def _sc_sparse_matmul(inp2, packed, bounds, bkg, rest10):
    mesh = plsc.VectorSubcoreMesh(core_axis_name="c", subcore_axis_name="s")

    @functools.partial(
        pl.kernel,
        out_type=jax.ShapeDtypeStruct((BS, N_OUT), jnp.float32),
        mesh=mesh,
        compiler_params=pltpu.CompilerParams(
            needs_layout_passes=False,
            use_tc_tiling_on_sc=False,
        ),
        scratch_types=[
            pltpu.VMEM((BSC // 2, N_IN), jnp.int32),
            pltpu.VMEM((BSC, RB), jnp.float32),
            pltpu.VMEM((3, CH), jnp.int32),
            pltpu.VMEM((3, CH), jnp.int32),
            pltpu.VMEM((NBOUNDS,), jnp.int32),
            pltpu.VMEM((RB,), jnp.float32),
            pltpu.VMEM((BSC, L), jnp.float32),
            pltpu.SemaphoreType.DMA,
            pltpu.SemaphoreType.DMA,
        ],
    )
    def body(inp_hbm, packed_hbm, bounds_hbm, bkg_hbm, rest_hbm,
             out_hbm, inp_v, acc_v, chA_v, chB_v, bounds_v, bkg_v,
             restm_v, semA, semB):
        wid = lax.axis_index("s") * 2 + lax.axis_index("c")
        half = wid % NHALF
        bs0 = (wid // NHALF) * BSC

        pltpu.sync_copy(inp_hbm.at[pl.ds(bs0 // 2, BSC // 2), :], inp_v)
        pltpu.sync_copy(rest_hbm.at[pl.ds(bs0, BSC), :], restm_v)
        pltpu.sync_copy(bounds_hbm, bounds_v)


        def pick(g):
            return bounds_v[pl.ds(g, L)][0]

        def block_body(b, _):
            g = half * BPH + b
            base = g * RB
            s = pick(g)
            e = pick(g + 1)

            pltpu.sync_copy(bkg_hbm.at[pl.ds(base, RB)], bkg_v)

            def init_body(r, _):
                bk = bkg_v[pl.ds(r * L, L)]
                for j in range(BSC):
                    acc_v[j, pl.ds(r * L, L)] = bk * restm_v[j, :]
                return 0

            lax.fori_loop(0, RB // L, init_body, 0)

            def process_chunk(buf, t, masked):

                @plsc.parallel_loop(0, NG, 1, unroll=2)
                def group_body(k):
                    off = k * L
                    rows16 = buf[0, pl.ds(off, L)]
                    cols16 = buf[1, pl.ds(off, L)]
                    w16 = plsc.bitcast(buf[2, pl.ds(off, L)], jnp.float32)
                    rloc = rows16 - base
                    if masked:
                        valid = (rows16 >= base) & (rows16 < base + RB)
                    else:
                        valid = None
                    for j2 in range(BSC // 2):
                        v = plsc.load_gather(inp_v.at[j2], [cols16])
                        ge = plsc.bitcast(lax.shift_left(v, 16),
                                          jnp.float32)
                        go = plsc.bitcast(v & jnp.int32(-65536),
                                          jnp.float32)
                        plsc.addupdate_scatter(acc_v.at[2 * j2], [rloc],
                                               ge * w16, mask=valid)
                        plsc.addupdate_scatter(acc_v.at[2 * j2 + 1], [rloc],
                                               go * w16, mask=valid)

            @pl.when(e > s)
            def _():
                t0 = s // CH
                t1 = (e - 1) // CH
                ntc = t1 - t0 + 1
                pltpu.async_copy(packed_hbm.at[t0], chA_v, semA)

                def chunk_loop(ci, _):
                    t = t0 + ci

                    @pl.when(ci % 2 == 0)
                    def _():
                        pltpu.make_async_copy(packed_hbm.at[t0], chA_v,
                                              semA).wait()

                        @pl.when(t + 1 <= t1)
                        def _():
                            pltpu.async_copy(packed_hbm.at[t + 1], chB_v,
                                             semB)

                        bnd = (t == t0) | (t == t1)

                        @pl.when(bnd)
                        def _():
                            process_chunk(chA_v, t, True)

                        @pl.when(jnp.logical_not(bnd))
                        def _():
                            process_chunk(chA_v, t, False)

                    @pl.when(ci % 2 == 1)
                    def _():
                        pltpu.make_async_copy(packed_hbm.at[t0], chB_v,
                                              semB).wait()

                        @pl.when(t + 1 <= t1)
                        def _():
                            pltpu.async_copy(packed_hbm.at[t + 1], chA_v,
                                             semA)

                        bnd = (t == t0) | (t == t1)

                        @pl.when(bnd)
                        def _():
                            process_chunk(chB_v, t, True)

                        @pl.when(jnp.logical_not(bnd))
                        def _():
                            process_chunk(chB_v, t, False)

                    return 0

                lax.fori_loop(0, ntc, chunk_loop, 0)

            pltpu.sync_copy(acc_v,
                            out_hbm.at[pl.ds(bs0, BSC), pl.ds(base, RB)])
            return 0

        lax.fori_loop(0, BPH, block_body, 0)

    return body(inp2, packed, bounds, bkg, rest10)


def kernel(inp, indices, weights, bkg_weights):
    b, s, f = inp.shape
    inp2 = inp.reshape(b * s, f).astype(jnp.float32)
    ev = lax.bitcast_convert_type(
        inp2[0::2].astype(jnp.bfloat16), jnp.uint16).astype(jnp.uint32)
    od = lax.bitcast_convert_type(
        inp2[1::2].astype(jnp.bfloat16), jnp.uint16).astype(jnp.uint32)
    inp_pk = lax.bitcast_convert_type(ev | (od << 16), jnp.int32)
    rows = indices[:, 0].astype(jnp.int32)
    cols = indices[:, 1].astype(jnp.int32)
    w32 = weights.astype(jnp.float32)

    nnz = rows.shape[0]
    pad = (-nnz) % CH
    if pad:
        rows_p = jnp.pad(rows, (0, pad), constant_values=N_OUT - 1)
        cols_p = jnp.pad(cols, (0, pad))
        w_p = jnp.pad(w32, (0, pad))
    else:
        rows_p, cols_p, w_p = rows, cols, w32
    lane = jnp.arange(CH)
    perm = (lane % L) * S + lane // L
    w_i = lax.bitcast_convert_type(w_p, jnp.int32)
    packed = jnp.stack([rows_p.reshape(-1, CH)[:, perm],
                        cols_p.reshape(-1, CH)[:, perm],
                        w_i.reshape(-1, CH)[:, perm]], axis=1)

    edges = jnp.arange(0, N_OUT + 1, RB, dtype=jnp.int32)
    bounds = jnp.searchsorted(rows, edges, side="left").astype(jnp.int32)
    bounds = jnp.pad(bounds, (0, NBOUNDS - bounds.shape[0]))

    kn = jax.random.key(42)
    rest = jnp.sum((jax.random.uniform(kn, (b, s, 10)) < 0.1)
                   .astype(jnp.float32), -1).reshape(b * s)
    rest10 = jnp.broadcast_to((rest / 10.0)[:, None], (b * s, L))

    out2 = _sc_sparse_matmul(inp_pk, packed, bounds,
                             bkg_weights.astype(jnp.float32), rest10)
    return out2.reshape(b, s, N_OUT)
